# baseline (speedup 1.0000x reference)
import functools

import jax
import jax.numpy as jnp
from jax import lax
from jax.experimental import pallas as pl
from jax.experimental.pallas import tpu as pltpu
from jax.experimental.pallas import tpu_sc as plsc


def _tc_body(x_ref, we_ref, be_ref, cb_ref, wd_ref, bd_ref,
             q_ref, loss_ref, table_ref):
    i = pl.program_id(0)
    cb = cb_ref[...]
    K = cb.shape[0]

    @pl.when(i == 0)
    def _():
        tbl = jnp.dot(cb, wd_ref[...],
                      preferred_element_type=jnp.float32) + bd_ref[...]
        pad = table_ref.shape[1] - tbl.shape[1]
        table_ref[...] = jnp.concatenate(
            [tbl, jnp.zeros((tbl.shape[0], pad), jnp.float32)], axis=1)

    x3 = x_ref[...]
    x = x3.reshape(x3.shape[0] * x3.shape[1], x3.shape[2])
    z = jnp.dot(x, we_ref[...], preferred_element_type=jnp.float32) + be_ref[...]
    zt = z.T
    csq = jnp.sum(cb * cb, axis=1)
    st = lax.dot_general(2.0 * cb, zt, (((1,), (0,)), ((), ())),
                         preferred_element_type=jnp.float32) - csq[:, None]
    G = K // 8
    s3 = st.reshape(G, 8, st.shape[1])
    best = s3[0]
    bj = jnp.zeros(best.shape, jnp.int32)
    for j in range(1, G):
        cur = s3[j]
        pred = cur > best
        best = jnp.where(pred, cur, best)
        bj = jnp.where(pred, jnp.int32(j), bj)
    sub = lax.broadcasted_iota(jnp.int32, best.shape, 0)
    cidx = bj * 8 + sub
    m = jnp.max(best, axis=0)
    q = jnp.min(jnp.where(best == m[None, :], cidx, jnp.int32(K)), axis=0)
    q_ref[0, 0, :] = q
    part = (jnp.sum(z * z) - jnp.sum(m)).reshape(1, 1)
    prev = jnp.where(i == 0, jnp.zeros((1, 1), jnp.float32), loss_ref[...])
    loss_ref[...] = prev + part


def _slice_body(src_ref, dst_ref):
    dst_ref[...] = src_ref[:, :, :dst_ref.shape[2]]


def _make_sc_gather(batch, seq, dpad):
    info = plsc.get_sparse_core_info()
    nc, ns = info.num_cores, info.num_subcores
    nw = nc * ns
    chunk = 128
    n_tokens = batch * seq
    per_w = n_tokens // nw
    n_ch = per_w // chunk
    assert per_w % chunk == 0 and seq % chunk == 0

    nbuf = 6

    @functools.partial(
        pl.kernel,
        mesh=plsc.VectorSubcoreMesh(core_axis_name="c", subcore_axis_name="s"),
        out_type=jax.ShapeDtypeStruct((batch, seq, dpad), jnp.float32),
        scratch_types=(
            [pltpu.VMEM((n_ch, chunk), jnp.int32),
             pltpu.VMEM((nbuf, chunk, dpad), jnp.float32)]
            + [pltpu.SemaphoreType.DMA] * (2 * nbuf)
        ),
    )
    def sc_gather(table_hbm, idx_hbm, out_hbm, idx_v, rows_v, *sems):
        gsem, ssem = sems[:nbuf], sems[nbuf:]
        wid = lax.axis_index("s") * nc + lax.axis_index("c")
        base = wid * per_w
        pltpu.sync_copy(idx_hbm.at[wid], idx_v)
        gcp = [None] * n_ch
        scp = [None] * n_ch

        def fire(c):
            gcp[c] = pltpu.async_copy(
                table_hbm.at[idx_v.at[c]], rows_v.at[c % nbuf], gsem[c % nbuf])

        for c in range(min(nbuf, n_ch)):
            fire(c)
        for c in range(n_ch):
            gcp[c].wait()
            t0 = base + c * chunk
            scp[c] = pltpu.async_copy(
                rows_v.at[c % nbuf],
                out_hbm.at[t0 // seq, pl.ds(t0 % seq, chunk)],
                ssem[c % nbuf])
            if c + nbuf < n_ch:
                scp[c].wait()
                fire(c + nbuf)
        for c in range(max(0, n_ch - nbuf), n_ch):
            if scp[c] is not None and c + nbuf >= n_ch:
                scp[c].wait()

    return sc_gather


def kernel(x, W_enc, b_enc, codebook, W_dec, b_dec):
    B, T, D_in = x.shape
    K, D_lat = codebook.shape
    N = B * T
    BLK = 2048
    DPAD = 128
    nblk = N // BLK

    BB = BLK // T
    q3, loss_sum, table = pl.pallas_call(
        _tc_body,
        grid=(nblk,),
        in_specs=[
            pl.BlockSpec((BB, T, D_in), lambda i: (i, 0, 0)),
            pl.BlockSpec((D_in, D_lat), lambda i: (0, 0)),
            pl.BlockSpec((1, D_lat), lambda i: (0, 0)),
            pl.BlockSpec((K, D_lat), lambda i: (0, 0)),
            pl.BlockSpec((D_lat, D_in), lambda i: (0, 0)),
            pl.BlockSpec((1, D_in), lambda i: (0, 0)),
        ],
        out_specs=[
            pl.BlockSpec((1, 1, BLK), lambda i: (i, 0, 0)),
            pl.BlockSpec((1, 1), lambda i: (0, 0)),
            pl.BlockSpec((K, DPAD), lambda i: (0, 0)),
        ],
        out_shape=[
            jax.ShapeDtypeStruct((nblk, 1, BLK), jnp.int32),
            jax.ShapeDtypeStruct((1, 1), jnp.float32),
            jax.ShapeDtypeStruct((K, DPAD), jnp.float32),
        ],
    )(x, W_enc, b_enc.reshape(1, D_lat), codebook, W_dec, b_dec.reshape(1, D_in))

    q = q3.reshape(N)
    nw = 32
    chunk = 128
    q_idx = q.reshape(nw, N // (nw * chunk), chunk)
    out_pad = _make_sc_gather(B, T, DPAD)(table, q_idx)

    SB = 4
    x_reco = pl.pallas_call(
        _slice_body,
        grid=(B // SB,),
        in_specs=[pl.BlockSpec((SB, T, DPAD), lambda i: (i, 0, 0))],
        out_specs=pl.BlockSpec((SB, T, D_in), lambda i: (i, 0, 0)),
        out_shape=jax.ShapeDtypeStruct((B, T, D_in), jnp.float32),
    )(out_pad)
    loss = (loss_sum[0, 0] / jnp.float32(N * D_lat)).reshape(())
    return (x_reco, loss, q.reshape(B, T))


# speedup vs baseline: 1.1236x; 1.1236x over previous
import functools

import jax
import jax.numpy as jnp
from jax import lax
from jax.experimental import pallas as pl
from jax.experimental.pallas import tpu as pltpu
from jax.experimental.pallas import tpu_sc as plsc


def _tc_body(x_ref, we_ref, be_ref, cb_ref, wd_ref, bd_ref,
             q_ref, loss_ref, table_ref):
    i = pl.program_id(0)
    cb = cb_ref[...]
    K = cb.shape[0]

    @pl.when(i == 0)
    def _():
        tbl = jnp.dot(cb, wd_ref[...],
                      preferred_element_type=jnp.float32) + bd_ref[...]
        pad = table_ref.shape[1] - tbl.shape[1]
        table_ref[...] = jnp.concatenate(
            [tbl, jnp.zeros((tbl.shape[0], pad), jnp.float32)], axis=1)

    x3 = x_ref[...]
    x = x3.reshape(x3.shape[0] * x3.shape[1], x3.shape[2])
    z = jnp.dot(x, we_ref[...], preferred_element_type=jnp.float32) + be_ref[...]
    zt = z.T
    csq = jnp.sum(cb * cb, axis=1)
    st = lax.dot_general(2.0 * cb, zt, (((1,), (0,)), ((), ())),
                         preferred_element_type=jnp.float32) - csq[:, None]
    G = K // 8
    s3 = st.reshape(G, 8, st.shape[1])
    best = s3[0]
    bj = jnp.zeros(best.shape, jnp.int32)
    for j in range(1, G):
        cur = s3[j]
        pred = cur > best
        best = jnp.where(pred, cur, best)
        bj = jnp.where(pred, jnp.int32(j), bj)
    sub = lax.broadcasted_iota(jnp.int32, best.shape, 0)
    cidx = bj * 8 + sub
    m = jnp.max(best, axis=0)
    q = jnp.min(jnp.where(best == m[None, :], cidx, jnp.int32(K)), axis=0)
    q_ref[0, 0, :] = q
    part = (jnp.sum(z * z) - jnp.sum(m)).reshape(1, 1)
    prev = jnp.where(i == 0, jnp.zeros((1, 1), jnp.float32), loss_ref[...])
    loss_ref[...] = prev + part


def _slice_body(src_ref, dst_ref):
    dst_ref[...] = src_ref[:, :, :dst_ref.shape[2]]


def _slice_alias_body(src_ref, prev_ref, dst_ref):
    del prev_ref
    dst_ref[...] = src_ref[:, :, :dst_ref.shape[2]]


def _make_sc_gather(batch, seq, dpad):
    info = plsc.get_sparse_core_info()
    nc, ns = info.num_cores, info.num_subcores
    nw = nc * ns
    chunk = 128
    n_tokens = batch * seq
    per_w = n_tokens // nw
    n_ch = per_w // chunk
    nbuf = 3
    assert per_w % chunk == 0 and seq % chunk == 0

    @functools.partial(
        pl.kernel,
        mesh=plsc.VectorSubcoreMesh(core_axis_name="c", subcore_axis_name="s"),
        out_type=jax.ShapeDtypeStruct((batch, seq, dpad), jnp.float32),
        scratch_types=(
            [pltpu.VMEM((n_ch, chunk), jnp.int32),
             pltpu.VMEM((nbuf, chunk, dpad), jnp.float32)]
            + [pltpu.SemaphoreType.DMA] * (2 * nbuf)
        ),
    )
    def sc_gather(table_hbm, idx_hbm, out_hbm, idx_v, rows_v, *sems):
        gsem, ssem = sems[:nbuf], sems[nbuf:]
        wid = lax.axis_index("s") * nc + lax.axis_index("c")
        base = wid * per_w
        pltpu.sync_copy(idx_hbm.at[wid], idx_v)
        gcp = [None] * n_ch
        scp = [None] * n_ch

        def fire(c):
            gcp[c] = pltpu.async_copy(
                table_hbm.at[idx_v.at[c]], rows_v.at[c % nbuf], gsem[c % nbuf])

        for c in range(min(nbuf, n_ch)):
            fire(c)
        for c in range(n_ch):
            gcp[c].wait()
            t0 = base + c * chunk
            scp[c] = pltpu.async_copy(
                rows_v.at[c % nbuf],
                out_hbm.at[t0 // seq, pl.ds(t0 % seq, chunk)],
                ssem[c % nbuf])
            if c + nbuf < n_ch:
                scp[c].wait()
                fire(c + nbuf)
        for c in range(n_ch):
            if c + nbuf >= n_ch:
                scp[c].wait()

    return sc_gather


def kernel(x, W_enc, b_enc, codebook, W_dec, b_dec):
    B, T, D_in = x.shape
    K, D_lat = codebook.shape
    N = B * T
    BLK = 2048
    DPAD = 128
    BB = BLK // T
    BH = B // 2
    nblk_h = BH // BB
    n_half = BH * T

    def tc_call(off):
        return pl.pallas_call(
            _tc_body,
            grid=(nblk_h,),
            in_specs=[
                pl.BlockSpec((BB, T, D_in), lambda i: (i + off, 0, 0)),
                pl.BlockSpec((D_in, D_lat), lambda i: (0, 0)),
                pl.BlockSpec((1, D_lat), lambda i: (0, 0)),
                pl.BlockSpec((K, D_lat), lambda i: (0, 0)),
                pl.BlockSpec((D_lat, D_in), lambda i: (0, 0)),
                pl.BlockSpec((1, D_in), lambda i: (0, 0)),
            ],
            out_specs=[
                pl.BlockSpec((1, 1, BLK), lambda i: (i, 0, 0)),
                pl.BlockSpec((1, 1), lambda i: (0, 0)),
                pl.BlockSpec((K, DPAD), lambda i: (0, 0)),
            ],
            out_shape=[
                jax.ShapeDtypeStruct((nblk_h, 1, BLK), jnp.int32),
                jax.ShapeDtypeStruct((1, 1), jnp.float32),
                jax.ShapeDtypeStruct((K, DPAD), jnp.float32),
            ],
        )(x, W_enc, b_enc.reshape(1, D_lat), codebook, W_dec,
          b_dec.reshape(1, D_in))

    qa3, loss_a, table = tc_call(0)
    qb3, loss_b, _ = tc_call(nblk_h)

    nw = 32
    gather = _make_sc_gather(BH, T, DPAD)
    pad_a = gather(table, qa3.reshape(nw, n_half // (nw * 128), 128))
    pad_b = gather(table, qb3.reshape(nw, n_half // (nw * 128), 128))

    SB = 4
    nsb = BH // SB
    half_out = pl.pallas_call(
        _slice_body,
        grid=(nsb,),
        in_specs=[pl.BlockSpec((SB, T, DPAD), lambda i: (i, 0, 0))],
        out_specs=pl.BlockSpec((SB, T, D_in), lambda i: (i, 0, 0)),
        out_shape=jax.ShapeDtypeStruct((B, T, D_in), jnp.float32),
    )(pad_a)
    x_reco = pl.pallas_call(
        _slice_alias_body,
        grid=(nsb,),
        in_specs=[
            pl.BlockSpec((SB, T, DPAD), lambda i: (i, 0, 0)),
            pl.BlockSpec(memory_space=pl.ANY),
        ],
        out_specs=pl.BlockSpec((SB, T, D_in), lambda i: (i + nsb, 0, 0)),
        out_shape=jax.ShapeDtypeStruct((B, T, D_in), jnp.float32),
        input_output_aliases={1: 0},
    )(pad_b, half_out)

    q = jnp.concatenate([qa3, qb3], axis=0).reshape(B, T)
    loss = ((loss_a[0, 0] + loss_b[0, 0]) / jnp.float32(N * D_lat)).reshape(())
    return (x_reco, loss, q)


# speedup vs baseline: 1.3846x; 1.2323x over previous
import functools

import jax
import jax.numpy as jnp
from jax import lax
from jax.experimental import pallas as pl
from jax.experimental.pallas import tpu as pltpu
from jax.experimental.pallas import tpu_sc as plsc


def _tc_body(x_ref, we_ref, be_ref, cb_ref, wd_ref, bd_ref,
             q_ref, loss_ref, table_ref):
    i = pl.program_id(0)
    cb = cb_ref[...]
    K = cb.shape[0]

    @pl.when(i == 0)
    def _():
        tbl = jnp.dot(cb, wd_ref[...],
                      preferred_element_type=jnp.float32) + bd_ref[...]
        pad = table_ref.shape[1] - tbl.shape[1]
        table_ref[...] = jnp.concatenate(
            [tbl, jnp.zeros((tbl.shape[0], pad), jnp.float32)], axis=1)

    x3 = x_ref[...]
    x = x3.reshape(x3.shape[0] * x3.shape[1], x3.shape[2])
    z = jnp.dot(x, we_ref[...], preferred_element_type=jnp.float32) + be_ref[...]
    zt = z.T
    csq = jnp.sum(cb * cb, axis=1)
    st = lax.dot_general(2.0 * cb, zt, (((1,), (0,)), ((), ())),
                         preferred_element_type=jnp.float32) - csq[:, None]
    G = K // 8
    s3 = st.reshape(G, 8, st.shape[1])
    best = s3[0]
    bj = jnp.zeros(best.shape, jnp.int32)
    for j in range(1, G):
        cur = s3[j]
        pred = cur > best
        best = jnp.where(pred, cur, best)
        bj = jnp.where(pred, jnp.int32(j), bj)
    sub = lax.broadcasted_iota(jnp.int32, best.shape, 0)
    cidx = bj * 8 + sub
    m = jnp.max(best, axis=0)
    q = jnp.min(jnp.where(best == m[None, :], cidx, jnp.int32(K)), axis=0)
    q_ref[0, 0, :] = q
    part = (jnp.sum(z * z) - jnp.sum(m)).reshape(1, 1)
    prev = jnp.where(i == 0, jnp.zeros((1, 1), jnp.float32), loss_ref[...])
    loss_ref[...] = prev + part


def _tc_body_dec(x_ref, we_ref, be_ref, cb_ref, wd_ref, bd_ref,
                 q_ref, loss_ref, xr_ref, tbl_ref):
    i = pl.program_id(0)
    cb = cb_ref[...]
    K = cb.shape[0]

    @pl.when(i == 0)
    def _():
        tbl_ref[...] = jnp.dot(cb, wd_ref[...],
                               preferred_element_type=jnp.float32) + bd_ref[...]

    x3 = x_ref[...]
    x = x3.reshape(x3.shape[0] * x3.shape[1], x3.shape[2])
    z = jnp.dot(x, we_ref[...], preferred_element_type=jnp.float32) + be_ref[...]
    zt = z.T
    csq = jnp.sum(cb * cb, axis=1)
    st = lax.dot_general(2.0 * cb, zt, (((1,), (0,)), ((), ())),
                         preferred_element_type=jnp.float32) - csq[:, None]
    G = K // 8
    s3 = st.reshape(G, 8, st.shape[1])
    best = s3[0]
    bj = jnp.zeros(best.shape, jnp.int32)
    for j in range(1, G):
        cur = s3[j]
        pred = cur > best
        best = jnp.where(pred, cur, best)
        bj = jnp.where(pred, jnp.int32(j), bj)
    sub = lax.broadcasted_iota(jnp.int32, best.shape, 0)
    cidx = bj * 8 + sub
    m = jnp.max(best, axis=0)
    q = jnp.min(jnp.where(best == m[None, :], cidx, jnp.int32(K)), axis=0)
    q_ref[0, 0, :] = q
    part = (jnp.sum(z * z) - jnp.sum(m)).reshape(1, 1)
    prev = jnp.where(i == 0, jnp.zeros((1, 1), jnp.float32), loss_ref[...])
    loss_ref[...] = prev + part

    lane = lax.broadcasted_iota(jnp.int32, (q.shape[0], K), 1)
    onehot = jnp.where(lane == q[:, None], jnp.float32(1.0), jnp.float32(0.0))
    xr = jnp.dot(onehot, tbl_ref[...], preferred_element_type=jnp.float32)
    xr_ref[...] = xr.reshape(xr_ref.shape)


def _slice_body(src_ref, dst_ref):
    dst_ref[...] = src_ref[:, :, :dst_ref.shape[2]]


def _slice_alias_body(src_ref, prev_ref, dst_ref):
    del prev_ref
    dst_ref[...] = src_ref[:, :, :dst_ref.shape[2]]


def _make_sc_gather(batch, seq, dpad):
    info = plsc.get_sparse_core_info()
    nc, ns = info.num_cores, info.num_subcores
    nw = nc * ns
    chunk = 128
    n_tokens = batch * seq
    per_w = n_tokens // nw
    n_ch = per_w // chunk
    nbuf = 3
    assert per_w % chunk == 0 and seq % chunk == 0

    @functools.partial(
        pl.kernel,
        mesh=plsc.VectorSubcoreMesh(core_axis_name="c", subcore_axis_name="s"),
        out_type=jax.ShapeDtypeStruct((batch, seq, dpad), jnp.float32),
        scratch_types=(
            [pltpu.VMEM((n_ch, chunk), jnp.int32),
             pltpu.VMEM((nbuf, chunk, dpad), jnp.float32)]
            + [pltpu.SemaphoreType.DMA] * (2 * nbuf)
        ),
    )
    def sc_gather(table_hbm, idx_hbm, out_hbm, idx_v, rows_v, *sems):
        gsem, ssem = sems[:nbuf], sems[nbuf:]
        wid = lax.axis_index("s") * nc + lax.axis_index("c")
        base = wid * per_w
        pltpu.sync_copy(idx_hbm.at[wid], idx_v)
        gcp = [None] * n_ch
        scp = [None] * n_ch

        def fire(c):
            gcp[c] = pltpu.async_copy(
                table_hbm.at[idx_v.at[c]], rows_v.at[c % nbuf], gsem[c % nbuf])

        for c in range(min(nbuf, n_ch)):
            fire(c)
        for c in range(n_ch):
            gcp[c].wait()
            t0 = base + c * chunk
            scp[c] = pltpu.async_copy(
                rows_v.at[c % nbuf],
                out_hbm.at[t0 // seq, pl.ds(t0 % seq, chunk)],
                ssem[c % nbuf])
            if c + nbuf < n_ch:
                scp[c].wait()
                fire(c + nbuf)
        for c in range(n_ch):
            if c + nbuf >= n_ch:
                scp[c].wait()

    return sc_gather


def kernel(x, W_enc, b_enc, codebook, W_dec, b_dec):
    B, T, D_in = x.shape
    K, D_lat = codebook.shape
    N = B * T
    BLK = 2048
    DPAD = 128
    BB = BLK // T
    BH = B // 2
    nblk_h = BH // BB
    n_half = BH * T

    nblk = B // BB
    B_SC = BH
    nblk_sc = B_SC // BB
    nblk_tc = nblk - nblk_sc
    n_sc = B_SC * T

    args = (x, W_enc, b_enc.reshape(1, D_lat), codebook, W_dec,
            b_dec.reshape(1, D_in))
    in_specs = [
        pl.BlockSpec((BB, T, D_in), lambda i: (i, 0, 0)),
        pl.BlockSpec((D_in, D_lat), lambda i: (0, 0)),
        pl.BlockSpec((1, D_lat), lambda i: (0, 0)),
        pl.BlockSpec((K, D_lat), lambda i: (0, 0)),
        pl.BlockSpec((D_lat, D_in), lambda i: (0, 0)),
        pl.BlockSpec((1, D_in), lambda i: (0, 0)),
    ]
    in_specs_b = list(in_specs)
    in_specs_b[0] = pl.BlockSpec((BB, T, D_in),
                                 lambda i: (i + nblk_tc, 0, 0))

    qb3, loss_b, table = pl.pallas_call(
        _tc_body,
        grid=(nblk_sc,),
        in_specs=in_specs_b,
        out_specs=[
            pl.BlockSpec((1, 1, BLK), lambda i: (i, 0, 0)),
            pl.BlockSpec((1, 1), lambda i: (0, 0)),
            pl.BlockSpec((K, DPAD), lambda i: (0, 0)),
        ],
        out_shape=[
            jax.ShapeDtypeStruct((nblk_sc, 1, BLK), jnp.int32),
            jax.ShapeDtypeStruct((1, 1), jnp.float32),
            jax.ShapeDtypeStruct((K, DPAD), jnp.float32),
        ],
    )(*args)

    nw = 32
    pad_b = _make_sc_gather(B_SC, T, DPAD)(
        table, qb3.reshape(nw, n_sc // (nw * 128), 128))

    qa3, loss_a, half_out = pl.pallas_call(
        _tc_body_dec,
        grid=(nblk_tc,),
        in_specs=in_specs,
        out_specs=[
            pl.BlockSpec((1, 1, BLK), lambda i: (i, 0, 0)),
            pl.BlockSpec((1, 1), lambda i: (0, 0)),
            pl.BlockSpec((BB, T, D_in), lambda i: (i, 0, 0)),
        ],
        out_shape=[
            jax.ShapeDtypeStruct((nblk_tc, 1, BLK), jnp.int32),
            jax.ShapeDtypeStruct((1, 1), jnp.float32),
            jax.ShapeDtypeStruct((B, T, D_in), jnp.float32),
        ],
        scratch_shapes=[pltpu.VMEM((K, D_in), jnp.float32)],
    )(*args)

    SB = 4
    nsb_tc = (B - B_SC) // SB
    x_reco = pl.pallas_call(
        _slice_alias_body,
        grid=(B_SC // SB,),
        in_specs=[
            pl.BlockSpec((SB, T, DPAD), lambda i: (i, 0, 0)),
            pl.BlockSpec(memory_space=pl.ANY),
        ],
        out_specs=pl.BlockSpec((SB, T, D_in), lambda i: (i + nsb_tc, 0, 0)),
        out_shape=jax.ShapeDtypeStruct((B, T, D_in), jnp.float32),
        input_output_aliases={1: 0},
    )(pad_b, half_out)

    q = jnp.concatenate([qa3, qb3], axis=0).reshape(B, T)
    loss = ((loss_a[0, 0] + loss_b[0, 0]) / jnp.float32(N * D_lat)).reshape(())
    return (x_reco, loss, q)


# speedup vs baseline: 1.5755x; 1.1379x over previous
import functools

import jax
import jax.numpy as jnp
from jax import lax
from jax.experimental import pallas as pl
from jax.experimental.pallas import tpu as pltpu
from jax.experimental.pallas import tpu_sc as plsc


def _tc_body(x_ref, we_ref, be_ref, cb_ref, wd_ref, bd_ref,
             q_ref, loss_ref, table_ref):
    i = pl.program_id(0)
    cb = cb_ref[...]
    K = cb.shape[0]

    @pl.when(i == 0)
    def _():
        tbl = jnp.dot(cb, wd_ref[...],
                      preferred_element_type=jnp.float32) + bd_ref[...]
        pad = table_ref.shape[1] - tbl.shape[1]
        table_ref[...] = jnp.concatenate(
            [tbl, jnp.zeros((tbl.shape[0], pad), jnp.float32)], axis=1)

    x3 = x_ref[...]
    x = x3.reshape(x3.shape[0] * x3.shape[1], x3.shape[2])
    z = jnp.dot(x, we_ref[...], preferred_element_type=jnp.float32) + be_ref[...]
    zt = z.T
    csq = jnp.sum(cb * cb, axis=1)
    st = lax.dot_general(2.0 * cb, zt, (((1,), (0,)), ((), ())),
                         preferred_element_type=jnp.float32) - csq[:, None]
    G = K // 8
    s3 = st.reshape(G, 8, st.shape[1])
    best = s3[0]
    bj = jnp.zeros(best.shape, jnp.int32)
    for j in range(1, G):
        cur = s3[j]
        pred = cur > best
        best = jnp.where(pred, cur, best)
        bj = jnp.where(pred, jnp.int32(j), bj)
    sub = lax.broadcasted_iota(jnp.int32, best.shape, 0)
    cidx = bj * 8 + sub
    m = jnp.max(best, axis=0)
    q = jnp.min(jnp.where(best == m[None, :], cidx, jnp.int32(K)), axis=0)
    q_ref[0, 0, :] = q
    part = (jnp.sum(z * z) - jnp.sum(m)).reshape(1, 1)
    prev = jnp.where(i == 0, jnp.zeros((1, 1), jnp.float32), loss_ref[...])
    loss_ref[...] = prev + part


def _tc_body_dec(x_ref, we_ref, be_ref, cb_ref, wd_ref, bd_ref,
                 q_ref, loss_ref, xr_ref, tbl_ref):
    i = pl.program_id(0)
    cb = cb_ref[...]
    K = cb.shape[0]

    @pl.when(i == 0)
    def _():
        tbl_ref[...] = jnp.dot(cb, wd_ref[...],
                               preferred_element_type=jnp.float32) + bd_ref[...]

    x3 = x_ref[...]
    x = x3.reshape(x3.shape[0] * x3.shape[1], x3.shape[2])
    z = jnp.dot(x, we_ref[...], preferred_element_type=jnp.float32) + be_ref[...]
    zt = z.T
    csq = jnp.sum(cb * cb, axis=1)
    st = lax.dot_general(2.0 * cb, zt, (((1,), (0,)), ((), ())),
                         preferred_element_type=jnp.float32) - csq[:, None]
    G = K // 8
    s3 = st.reshape(G, 8, st.shape[1])
    best = s3[0]
    bj = jnp.zeros(best.shape, jnp.int32)
    for j in range(1, G):
        cur = s3[j]
        pred = cur > best
        best = jnp.where(pred, cur, best)
        bj = jnp.where(pred, jnp.int32(j), bj)
    sub = lax.broadcasted_iota(jnp.int32, best.shape, 0)
    cidx = bj * 8 + sub
    m = jnp.max(best, axis=0)
    q = jnp.min(jnp.where(best == m[None, :], cidx, jnp.int32(K)), axis=0)
    q_ref[0, 0, :] = q
    part = (jnp.sum(z * z) - jnp.sum(m)).reshape(1, 1)
    prev = jnp.where(i == 0, jnp.zeros((1, 1), jnp.float32), loss_ref[...])
    loss_ref[...] = prev + part

    lane = lax.broadcasted_iota(jnp.int32, (q.shape[0], K), 1)
    onehot = jnp.where(lane == q[:, None], jnp.float32(1.0), jnp.float32(0.0))
    xr = jnp.dot(onehot, tbl_ref[...], preferred_element_type=jnp.float32)
    xr_ref[...] = xr.reshape(xr_ref.shape)


def _slice_body(src_ref, dst_ref):
    dst_ref[...] = src_ref[:, :, :dst_ref.shape[2]]


def _slice_alias_body(src_ref, prev_ref, dst_ref):
    del prev_ref
    dst_ref[...] = src_ref[:, :, :dst_ref.shape[2]]


def _make_sc_gather(batch, seq, dpad):
    info = plsc.get_sparse_core_info()
    nc, ns = info.num_cores, info.num_subcores
    nw = nc * ns
    chunk = 128
    n_tokens = batch * seq
    per_w = n_tokens // nw
    n_ch = per_w // chunk
    nbuf = 3
    assert per_w % chunk == 0 and seq % chunk == 0

    @functools.partial(
        pl.kernel,
        mesh=plsc.VectorSubcoreMesh(core_axis_name="c", subcore_axis_name="s"),
        out_type=jax.ShapeDtypeStruct((batch, seq, dpad), jnp.float32),
        scratch_types=(
            [pltpu.VMEM((n_ch, chunk), jnp.int32),
             pltpu.VMEM((nbuf, chunk, dpad), jnp.float32)]
            + [pltpu.SemaphoreType.DMA] * (2 * nbuf)
        ),
    )
    def sc_gather(table_hbm, idx_hbm, out_hbm, idx_v, rows_v, *sems):
        gsem, ssem = sems[:nbuf], sems[nbuf:]
        wid = lax.axis_index("s") * nc + lax.axis_index("c")
        base = wid * per_w
        pltpu.sync_copy(idx_hbm.at[wid], idx_v)
        gcp = [None] * n_ch
        scp = [None] * n_ch

        def fire(c):
            gcp[c] = pltpu.async_copy(
                table_hbm.at[idx_v.at[c]], rows_v.at[c % nbuf], gsem[c % nbuf])

        for c in range(min(nbuf, n_ch)):
            fire(c)
        for c in range(n_ch):
            gcp[c].wait()
            t0 = base + c * chunk
            scp[c] = pltpu.async_copy(
                rows_v.at[c % nbuf],
                out_hbm.at[t0 // seq, pl.ds(t0 % seq, chunk)],
                ssem[c % nbuf])
            if c + nbuf < n_ch:
                scp[c].wait()
                fire(c + nbuf)
        for c in range(n_ch):
            if c + nbuf >= n_ch:
                scp[c].wait()

    return sc_gather


def kernel(x, W_enc, b_enc, codebook, W_dec, b_dec):
    B, T, D_in = x.shape
    K, D_lat = codebook.shape
    N = B * T
    BLK = 2048
    DPAD = 128
    BB = BLK // T
    BH = B // 2
    nblk_h = BH // BB
    n_half = BH * T

    nblk = B // BB
    B_SC = B // 4
    nblk_sc = B_SC // BB
    nblk_tc = nblk - nblk_sc
    n_sc = B_SC * T

    args = (x, W_enc, b_enc.reshape(1, D_lat), codebook, W_dec,
            b_dec.reshape(1, D_in))
    in_specs = [
        pl.BlockSpec((BB, T, D_in), lambda i: (i, 0, 0)),
        pl.BlockSpec((D_in, D_lat), lambda i: (0, 0)),
        pl.BlockSpec((1, D_lat), lambda i: (0, 0)),
        pl.BlockSpec((K, D_lat), lambda i: (0, 0)),
        pl.BlockSpec((D_lat, D_in), lambda i: (0, 0)),
        pl.BlockSpec((1, D_in), lambda i: (0, 0)),
    ]
    in_specs_b = list(in_specs)
    in_specs_b[0] = pl.BlockSpec((BB, T, D_in),
                                 lambda i: (i + nblk_tc, 0, 0))

    qb3, loss_b, table = pl.pallas_call(
        _tc_body,
        grid=(nblk_sc,),
        in_specs=in_specs_b,
        out_specs=[
            pl.BlockSpec((1, 1, BLK), lambda i: (i, 0, 0)),
            pl.BlockSpec((1, 1), lambda i: (0, 0)),
            pl.BlockSpec((K, DPAD), lambda i: (0, 0)),
        ],
        out_shape=[
            jax.ShapeDtypeStruct((nblk_sc, 1, BLK), jnp.int32),
            jax.ShapeDtypeStruct((1, 1), jnp.float32),
            jax.ShapeDtypeStruct((K, DPAD), jnp.float32),
        ],
    )(*args)

    nw = 32
    pad_b = _make_sc_gather(B_SC, T, DPAD)(
        table, qb3.reshape(nw, n_sc // (nw * 128), 128))

    qa3, loss_a, half_out = pl.pallas_call(
        _tc_body_dec,
        grid=(nblk_tc,),
        in_specs=in_specs,
        out_specs=[
            pl.BlockSpec((1, 1, BLK), lambda i: (i, 0, 0)),
            pl.BlockSpec((1, 1), lambda i: (0, 0)),
            pl.BlockSpec((BB, T, D_in), lambda i: (i, 0, 0)),
        ],
        out_shape=[
            jax.ShapeDtypeStruct((nblk_tc, 1, BLK), jnp.int32),
            jax.ShapeDtypeStruct((1, 1), jnp.float32),
            jax.ShapeDtypeStruct((B, T, D_in), jnp.float32),
        ],
        scratch_shapes=[pltpu.VMEM((K, D_in), jnp.float32)],
    )(*args)

    SB = 4
    nsb_tc = (B - B_SC) // SB
    x_reco = pl.pallas_call(
        _slice_alias_body,
        grid=(B_SC // SB,),
        in_specs=[
            pl.BlockSpec((SB, T, DPAD), lambda i: (i, 0, 0)),
            pl.BlockSpec(memory_space=pl.ANY),
        ],
        out_specs=pl.BlockSpec((SB, T, D_in), lambda i: (i + nsb_tc, 0, 0)),
        out_shape=jax.ShapeDtypeStruct((B, T, D_in), jnp.float32),
        input_output_aliases={1: 0},
    )(pad_b, half_out)

    q = jnp.concatenate([qa3, qb3], axis=0).reshape(B, T)
    loss = ((loss_a[0, 0] + loss_b[0, 0]) / jnp.float32(N * D_lat)).reshape(())
    return (x_reco, loss, q)


# speedup vs baseline: 1.6551x; 1.0505x over previous
import functools

import jax
import jax.numpy as jnp
from jax import lax
from jax.experimental import pallas as pl
from jax.experimental.pallas import tpu as pltpu
from jax.experimental.pallas import tpu_sc as plsc


def _tc_body(x_ref, we_ref, be_ref, cb_ref, wd_ref, bd_ref,
             q_ref, loss_ref, table_ref):
    i = pl.program_id(0)
    cb = cb_ref[...]
    K = cb.shape[0]

    @pl.when(i == 0)
    def _():
        tbl = jnp.dot(cb, wd_ref[...],
                      preferred_element_type=jnp.float32) + bd_ref[...]
        pad = table_ref.shape[1] - tbl.shape[1]
        table_ref[...] = jnp.concatenate(
            [tbl, jnp.zeros((tbl.shape[0], pad), jnp.float32)], axis=1)

    x3 = x_ref[...]
    x = x3.reshape(x3.shape[0] * x3.shape[1], x3.shape[2])
    z = jnp.dot(x, we_ref[...], preferred_element_type=jnp.float32) + be_ref[...]
    zt = z.T
    csq = jnp.sum(cb * cb, axis=1)
    st = lax.dot_general(2.0 * cb, zt, (((1,), (0,)), ((), ())),
                         preferred_element_type=jnp.float32) - csq[:, None]
    G = K // 8
    s3 = st.reshape(G, 8, st.shape[1])
    best = s3[0]
    bj = jnp.zeros(best.shape, jnp.int32)
    for j in range(1, G):
        cur = s3[j]
        pred = cur > best
        best = jnp.where(pred, cur, best)
        bj = jnp.where(pred, jnp.int32(j), bj)
    sub = lax.broadcasted_iota(jnp.int32, best.shape, 0)
    cidx = bj * 8 + sub
    m = jnp.max(best, axis=0)
    q = jnp.min(jnp.where(best == m[None, :], cidx, jnp.int32(K)), axis=0)
    q_ref[0, 0, :] = q
    part = (jnp.sum(z * z) - jnp.sum(m)).reshape(1, 1)
    prev = jnp.where(i == 0, jnp.zeros((1, 1), jnp.float32), loss_ref[...])
    loss_ref[...] = prev + part


def _tc_body_dec(x_ref, we_ref, be_ref, cb_ref, wd_ref, bd_ref,
                 q_ref, loss_ref, xr_ref, tbl_ref):
    i = pl.program_id(0)
    cb = cb_ref[...]
    K = cb.shape[0]

    @pl.when(i == 0)
    def _():
        tbl_ref[...] = jnp.dot(cb, wd_ref[...],
                               preferred_element_type=jnp.float32) + bd_ref[...]

    x3 = x_ref[...]
    x = x3.reshape(x3.shape[0] * x3.shape[1], x3.shape[2])
    z = jnp.dot(x, we_ref[...], preferred_element_type=jnp.float32) + be_ref[...]
    zt = z.T
    csq = jnp.sum(cb * cb, axis=1)
    st = lax.dot_general(2.0 * cb, zt, (((1,), (0,)), ((), ())),
                         preferred_element_type=jnp.float32) - csq[:, None]
    G = K // 8
    s3 = st.reshape(G, 8, st.shape[1])
    best = s3[0]
    bj = jnp.zeros(best.shape, jnp.int32)
    for j in range(1, G):
        cur = s3[j]
        pred = cur > best
        best = jnp.where(pred, cur, best)
        bj = jnp.where(pred, jnp.int32(j), bj)
    sub = lax.broadcasted_iota(jnp.int32, best.shape, 0)
    cidx = bj * 8 + sub
    m = jnp.max(best, axis=0)
    q = jnp.min(jnp.where(best == m[None, :], cidx, jnp.int32(K)), axis=0)
    q_ref[0, 0, :] = q
    part = (jnp.sum(z * z) - jnp.sum(m)).reshape(1, 1)
    prev = jnp.where(i == 0, jnp.zeros((1, 1), jnp.float32), loss_ref[...])
    loss_ref[...] = prev + part

    lane = lax.broadcasted_iota(jnp.int32, (q.shape[0], K), 1)
    onehot = jnp.where(lane == q[:, None], jnp.float32(1.0), jnp.float32(0.0))
    xr = jnp.dot(onehot, tbl_ref[...], preferred_element_type=jnp.float32)
    xr_ref[...] = xr.reshape(xr_ref.shape)


def _slice_body(src_ref, dst_ref):
    dst_ref[...] = src_ref[:, :, :dst_ref.shape[2]]


def _slice_alias_body(src_ref, prev_ref, dst_ref):
    del prev_ref
    dst_ref[...] = src_ref[:, :, :dst_ref.shape[2]]


def _make_sc_gather(batch, seq, dpad):
    info = plsc.get_sparse_core_info()
    nc, ns = info.num_cores, info.num_subcores
    nw = nc * ns
    chunk = 128
    n_tokens = batch * seq
    per_w = n_tokens // nw
    n_ch = per_w // chunk
    nbuf = 3
    assert per_w % chunk == 0 and seq % chunk == 0

    @functools.partial(
        pl.kernel,
        mesh=plsc.VectorSubcoreMesh(core_axis_name="c", subcore_axis_name="s"),
        out_type=jax.ShapeDtypeStruct((batch, seq, dpad), jnp.float32),
        scratch_types=(
            [pltpu.VMEM((n_ch, chunk), jnp.int32),
             pltpu.VMEM((nbuf, chunk, dpad), jnp.float32)]
            + [pltpu.SemaphoreType.DMA] * (2 * nbuf)
        ),
    )
    def sc_gather(table_hbm, idx_hbm, out_hbm, idx_v, rows_v, *sems):
        gsem, ssem = sems[:nbuf], sems[nbuf:]
        wid = lax.axis_index("s") * nc + lax.axis_index("c")
        base = wid * per_w
        pltpu.sync_copy(idx_hbm.at[wid], idx_v)
        gcp = [None] * n_ch
        scp = [None] * n_ch

        def fire(c):
            gcp[c] = pltpu.async_copy(
                table_hbm.at[idx_v.at[c]], rows_v.at[c % nbuf], gsem[c % nbuf])

        for c in range(min(nbuf, n_ch)):
            fire(c)
        for c in range(n_ch):
            gcp[c].wait()
            t0 = base + c * chunk
            scp[c] = pltpu.async_copy(
                rows_v.at[c % nbuf],
                out_hbm.at[t0 // seq, pl.ds(t0 % seq, chunk)],
                ssem[c % nbuf])
            if c + nbuf < n_ch:
                scp[c].wait()
                fire(c + nbuf)
        for c in range(n_ch):
            if c + nbuf >= n_ch:
                scp[c].wait()

    return sc_gather


def kernel(x, W_enc, b_enc, codebook, W_dec, b_dec):
    B, T, D_in = x.shape
    K, D_lat = codebook.shape
    N = B * T
    BLK = 2048
    DPAD = 128
    BB = BLK // T
    BH = B // 2
    nblk_h = BH // BB
    n_half = BH * T

    nblk = B // BB
    B_SC = B // 8
    nblk_sc = B_SC // BB
    nblk_tc = nblk - nblk_sc
    n_sc = B_SC * T

    args = (x, W_enc, b_enc.reshape(1, D_lat), codebook, W_dec,
            b_dec.reshape(1, D_in))
    in_specs = [
        pl.BlockSpec((BB, T, D_in), lambda i: (i, 0, 0)),
        pl.BlockSpec((D_in, D_lat), lambda i: (0, 0)),
        pl.BlockSpec((1, D_lat), lambda i: (0, 0)),
        pl.BlockSpec((K, D_lat), lambda i: (0, 0)),
        pl.BlockSpec((D_lat, D_in), lambda i: (0, 0)),
        pl.BlockSpec((1, D_in), lambda i: (0, 0)),
    ]
    in_specs_b = list(in_specs)
    in_specs_b[0] = pl.BlockSpec((BB, T, D_in),
                                 lambda i: (i + nblk_tc, 0, 0))

    qb3, loss_b, table = pl.pallas_call(
        _tc_body,
        grid=(nblk_sc,),
        in_specs=in_specs_b,
        out_specs=[
            pl.BlockSpec((1, 1, BLK), lambda i: (i, 0, 0)),
            pl.BlockSpec((1, 1), lambda i: (0, 0)),
            pl.BlockSpec((K, DPAD), lambda i: (0, 0)),
        ],
        out_shape=[
            jax.ShapeDtypeStruct((nblk_sc, 1, BLK), jnp.int32),
            jax.ShapeDtypeStruct((1, 1), jnp.float32),
            jax.ShapeDtypeStruct((K, DPAD), jnp.float32),
        ],
    )(*args)

    nw = 32
    pad_b = _make_sc_gather(B_SC, T, DPAD)(
        table, qb3.reshape(nw, n_sc // (nw * 128), 128))

    qa3, loss_a, half_out = pl.pallas_call(
        _tc_body_dec,
        grid=(nblk_tc,),
        in_specs=in_specs,
        out_specs=[
            pl.BlockSpec((1, 1, BLK), lambda i: (i, 0, 0)),
            pl.BlockSpec((1, 1), lambda i: (0, 0)),
            pl.BlockSpec((BB, T, D_in), lambda i: (i, 0, 0)),
        ],
        out_shape=[
            jax.ShapeDtypeStruct((nblk_tc, 1, BLK), jnp.int32),
            jax.ShapeDtypeStruct((1, 1), jnp.float32),
            jax.ShapeDtypeStruct((B, T, D_in), jnp.float32),
        ],
        scratch_shapes=[pltpu.VMEM((K, D_in), jnp.float32)],
    )(*args)

    SB = 4
    nsb_tc = (B - B_SC) // SB
    x_reco = pl.pallas_call(
        _slice_alias_body,
        grid=(B_SC // SB,),
        in_specs=[
            pl.BlockSpec((SB, T, DPAD), lambda i: (i, 0, 0)),
            pl.BlockSpec(memory_space=pl.ANY),
        ],
        out_specs=pl.BlockSpec((SB, T, D_in), lambda i: (i + nsb_tc, 0, 0)),
        out_shape=jax.ShapeDtypeStruct((B, T, D_in), jnp.float32),
        input_output_aliases={1: 0},
    )(pad_b, half_out)

    q = jnp.concatenate([qa3, qb3], axis=0).reshape(B, T)
    loss = ((loss_a[0, 0] + loss_b[0, 0]) / jnp.float32(N * D_lat)).reshape(())
    return (x_reco, loss, q)
